# Initial kernel scaffold; baseline (speedup 1.0000x reference)
import jax
import jax.numpy as jnp
from jax.experimental import pallas as pl


def kernel(x, W_g, W_up, W_down):
    raise NotImplementedError("write your pallas kernel here")


# speedup vs baseline: 2.4409x; 2.4409x over previous
import functools

import jax
import jax.numpy as jnp
from jax.experimental import pallas as pl


def _contract_last(a, b):
    return jax.lax.dot_general(
        a, b, (((1,), (1,)), ((), ())), preferred_element_type=jnp.float32
    )


def _moe_kernel(x_ref, wg_ref, wup_ref, wdown_ref, o_ref):
    j = pl.program_id(1)
    nj = pl.num_programs(1)

    x = x_ref[...]
    h = _contract_last(x, wup_ref[...])
    h = h * jax.nn.sigmoid(h)
    contrib = _contract_last(h.astype(jnp.bfloat16), wdown_ref[...])

    @pl.when(j == 0)
    def _init():
        o_ref[...] = contrib

    @pl.when(jnp.logical_and(j > 0, j < nj - 1))
    def _acc():
        o_ref[...] += contrib

    @pl.when(j == nj - 1)
    def _final():
        acc = o_ref[...] + contrib
        g = _contract_last(x, wg_ref[...])
        m = jnp.max(g, axis=1, keepdims=True)
        e = jnp.exp(g - m)
        z = jnp.sum(e, axis=1, keepdims=True)
        m1 = jnp.max(e, axis=1, keepdims=True)
        iota = jax.lax.broadcasted_iota(jnp.int32, g.shape, 1)
        first = jnp.min(
            jnp.where(e == m1, iota, g.shape[1]), axis=1, keepdims=True
        )
        e2 = jnp.where(iota == first, 0.0, e)
        m2 = jnp.max(e2, axis=1, keepdims=True)
        s = (m1 + m2) / z
        scale = s / (s + 1e-9)
        o_ref[...] = acc * scale


@functools.partial(jax.jit, static_argnames=("tm", "te"))
def _run(xf, wg, wup, wdown, tm, te):
    t, d = xf.shape
    ed = wup.shape[0]
    grid = (t // tm, ed // te)
    return pl.pallas_call(
        _moe_kernel,
        grid=grid,
        in_specs=[
            pl.BlockSpec((tm, d), lambda i, j: (i, 0)),
            pl.BlockSpec(wg.shape, lambda i, j: (0, 0)),
            pl.BlockSpec((te, d), lambda i, j: (j, 0)),
            pl.BlockSpec((d, te), lambda i, j: (0, j)),
        ],
        out_specs=pl.BlockSpec((tm, d), lambda i, j: (i, 0)),
        out_shape=jax.ShapeDtypeStruct((t, d), jnp.float32),
    )(xf, wg, wup, wdown)


def kernel(x, W_g, W_up, W_down):
    b, s, d = x.shape
    xf = x.reshape(b * s, d).astype(jnp.bfloat16)
    out = _run(
        xf,
        W_g.astype(jnp.bfloat16),
        W_up.astype(jnp.bfloat16),
        W_down.astype(jnp.bfloat16),
        tm=512,
        te=512,
    )
    return out.reshape(b, s, d)


# speedup vs baseline: 2.9649x; 1.2147x over previous
import functools

import jax
import jax.numpy as jnp
from jax.experimental import pallas as pl


def _contract_last(a, b):
    return jax.lax.dot_general(
        a, b, (((1,), (1,)), ((), ())), preferred_element_type=jnp.float32
    )


def _moe_kernel(x_ref, wg_ref, wup_ref, wdown_ref, o_ref):
    x = x_ref[...]
    h = _contract_last(x, wup_ref[...])
    h = h * jax.nn.sigmoid(h)
    out = _contract_last(h.astype(jnp.bfloat16), wdown_ref[...])
    g = _contract_last(x, wg_ref[...])
    m = jnp.max(g, axis=1, keepdims=True)
    e = jnp.exp(g - m)
    z = jnp.sum(e, axis=1, keepdims=True)
    m1 = jnp.max(e, axis=1, keepdims=True)
    iota = jax.lax.broadcasted_iota(jnp.int32, g.shape, 1)
    first = jnp.min(
        jnp.where(e == m1, iota, g.shape[1]), axis=1, keepdims=True
    )
    e2 = jnp.where(iota == first, 0.0, e)
    m2 = jnp.max(e2, axis=1, keepdims=True)
    s = (m1 + m2) / z
    scale = s / (s + 1e-9)
    o_ref[...] = out * scale


@functools.partial(jax.jit, static_argnames=("tm",))
def _run(xf, wg, wup, wdown, tm):
    t, d = xf.shape
    ed = wup.shape[0]
    return pl.pallas_call(
        _moe_kernel,
        grid=(t // tm,),
        in_specs=[
            pl.BlockSpec((tm, d), lambda i: (i, 0)),
            pl.BlockSpec(wg.shape, lambda i: (0, 0)),
            pl.BlockSpec((ed, d), lambda i: (0, 0)),
            pl.BlockSpec((d, ed), lambda i: (0, 0)),
        ],
        out_specs=pl.BlockSpec((tm, d), lambda i: (i, 0)),
        out_shape=jax.ShapeDtypeStruct((t, d), jnp.float32),
    )(xf, wg, wup, wdown)


def kernel(x, W_g, W_up, W_down):
    b, s, d = x.shape
    xf = x.reshape(b * s, d).astype(jnp.bfloat16)
    out = _run(
        xf,
        W_g.astype(jnp.bfloat16),
        W_up.astype(jnp.bfloat16),
        W_down.astype(jnp.bfloat16),
        tm=256,
    )
    return out.reshape(b, s, d)
